# baseline (speedup 1.0000x reference)
import functools

import jax
import jax.numpy as jnp
from jax import lax
from jax.experimental import pallas as pl
from jax.experimental.pallas import tpu as pltpu
from jax.experimental.pallas import tpu_sc as plsc

NW = 32
LANES = 16


def _worker_id():
    c = lax.axis_index("c")
    s = lax.axis_index("s")
    return s * 2 + c


def _make_deg(N, E, CH):
    PER = E // NW
    assert CH % LANES == 0 and N % LANES == 0 and PER % CH == 0
    mesh = plsc.VectorSubcoreMesh(core_axis_name="c", subcore_axis_name="s")

    @functools.partial(
        pl.kernel,
        out_type=[
            jax.ShapeDtypeStruct((NW, N), jnp.float32),
            jax.ShapeDtypeStruct((E,), jnp.int32),
        ],
        mesh=mesh,
        compiler_params=pltpu.CompilerParams(needs_layout_passes=False),
        scratch_types=[
            pltpu.VMEM((N,), jnp.float32),
            pltpu.VMEM((CH,), jnp.int32),
            pltpu.VMEM((CH,), jnp.int32),
            pltpu.VMEM((CH,), jnp.float32),
            pltpu.VMEM((CH,), jnp.int32),
            pltpu.SemaphoreType.DMA,
        ],
    )
    def deg_kernel(src, dst, ew, degp, pk, acc, src_v, dst_v, ew_v, pk_v, sem):
        wid = _worker_id()
        ebase = wid * PER
        copies = [
            pltpu.make_async_copy(src.at[pl.ds(ebase, CH)], src_v, sem),
            pltpu.make_async_copy(dst.at[pl.ds(ebase, CH)], dst_v, sem),
            pltpu.make_async_copy(ew.at[pl.ds(ebase, CH)], ew_v, sem),
        ]
        for cp in copies:
            cp.start()

        zero = jnp.zeros((LANES,), jnp.float32)

        @plsc.parallel_loop(0, N, LANES, unroll=8)
        def _zero(i):
            acc[pl.ds(i, LANES)] = zero

        for cp in copies:
            cp.wait()

        @plsc.parallel_loop(0, CH, LANES, unroll=8)
        def _grp(i):
            sv = src_v[pl.ds(i, LANES)]
            dv = dst_v[pl.ds(i, LANES)]
            wv = ew_v[pl.ds(i, LANES)]
            pk_v[pl.ds(i, LANES)] = jnp.bitwise_or(
                jnp.left_shift(dv, 16), sv)
            plsc.addupdate_scatter(acc, [dv], wv)

        pltpu.sync_copy(pk_v, pk.at[pl.ds(ebase, CH)])
        pltpu.sync_copy(acc, degp.at[wid])

    return deg_kernel


def _make_agg(N, E, F, FB, G, CH):
    NFB = F // FB
    assert G * NFB == NW
    assert CH % LANES == 0 and N % LANES == 0
    PER = E // G
    NCH = PER // CH
    NB = 2 if NCH > 1 else 1
    assert NCH % NB == 0
    mesh = plsc.VectorSubcoreMesh(core_axis_name="c", subcore_axis_name="s")
    scratch = (
        [pltpu.VMEM((N,), jnp.float32) for _ in range(2 * FB)]
        + [pltpu.VMEM((CH,), jnp.int32) for _ in range(NB)]
        + [pltpu.VMEM((CH,), jnp.float32) for _ in range(NB)]
        + [pltpu.SemaphoreType.DMA for _ in range(NB + 1)]
    )

    @functools.partial(
        pl.kernel,
        out_type=jax.ShapeDtypeStruct((G, F, N), jnp.float32),
        mesh=mesh,
        compiler_params=pltpu.CompilerParams(needs_layout_passes=False),
        scratch_types=scratch,
    )
    def agg_kernel(hT, pk, ew, part, *scr):
        tbls = scr[:FB]
        accs = scr[FB:2 * FB]
        pk_v = scr[2 * FB:2 * FB + NB]
        ew_v = scr[2 * FB + NB:2 * FB + 2 * NB]
        sems = scr[2 * FB + 2 * NB:2 * FB + 3 * NB]
        tsem = scr[2 * FB + 3 * NB]
        wid = _worker_id()
        g = wid % G
        fb = wid // G
        fbase = fb * FB
        ebase = g * PER

        def edge_copies(ci, b):
            off = ebase + ci * CH
            return [
                pltpu.make_async_copy(pk.at[pl.ds(off, CH)], pk_v[b], sems[b]),
                pltpu.make_async_copy(ew.at[pl.ds(off, CH)], ew_v[b], sems[b]),
            ]

        def issue(ci, b):
            for cp in edge_copies(ci, b):
                cp.start()

        def wait(ci, b):
            for cp in edge_copies(ci, b):
                cp.wait()

        for b in range(NB):
            issue(b, b)
        tcopies = [
            pltpu.make_async_copy(hT.at[fbase + k], tbls[k], tsem)
            for k in range(FB)
        ]
        for cp in tcopies:
            cp.start()

        zero = jnp.zeros((LANES,), jnp.float32)

        @plsc.parallel_loop(0, N, LANES, unroll=4)
        def _zero(i):
            for k in range(FB):
                accs[k][pl.ds(i, LANES)] = zero

        for cp in tcopies:
            cp.wait()

        def process(b):
            @plsc.parallel_loop(0, CH, LANES, unroll=8)
            def _grp(i):
                pkv = pk_v[b][pl.ds(i, LANES)]
                sv = jnp.bitwise_and(pkv, jnp.int32(0xFFFF))
                dv = jnp.right_shift(pkv, 16)
                wv = ew_v[b][pl.ds(i, LANES)]
                for k in range(FB):
                    hv = plsc.load_gather(tbls[k], [sv])
                    plsc.addupdate_scatter(accs[k], [dv], hv * wv)

        if NB == 1:
            def chunk(ci, _):
                wait(ci, 0)
                process(0)

                @pl.when(ci + 1 < NCH)
                def _():
                    issue(ci + 1, 0)
                return 0

            lax.fori_loop(0, NCH, chunk, 0)
        else:
            def pair(pi, _):
                ci0 = pi * 2
                for b in range(2):
                    wait(ci0 + b, b)
                    process(b)

                    @pl.when(ci0 + b + 2 < NCH)
                    def _():
                        issue(ci0 + b + 2, b)
                return 0

            lax.fori_loop(0, NCH // 2, pair, 0)

        for k in range(FB):
            pltpu.sync_copy(accs[k], part.at[g, fbase + k])

    return agg_kernel


def _tc_head(degp, x, W1, BN):
    NWp, N = degp.shape
    D = x.shape[1]
    H = W1.shape[1]
    grid = (pl.cdiv(N, BN),)

    def body(degp_ref, x_ref, w_ref, zp_ref, dinv_ref):
        deg = jnp.sum(degp_ref[...], axis=0, keepdims=True) + 1.0
        dinv = lax.rsqrt(deg)
        z = lax.dot_general(w_ref[...], x_ref[...], (((0,), (1,)), ((), ())),
                            preferred_element_type=jnp.float32)
        zp_ref[...] = z * dinv
        dinv_ref[...] = dinv

    return pl.pallas_call(
        body,
        grid=grid,
        in_specs=[
            pl.BlockSpec((NWp, BN), lambda i: (0, i)),
            pl.BlockSpec((BN, D), lambda i: (i, 0)),
            pl.BlockSpec((D, H), lambda i: (0, 0)),
        ],
        out_specs=[
            pl.BlockSpec((H, BN), lambda i: (0, i)),
            pl.BlockSpec((1, BN), lambda i: (0, i)),
        ],
        out_shape=[
            jax.ShapeDtypeStruct((H, N), jnp.float32),
            jax.ShapeDtypeStruct((1, N), jnp.float32),
        ],
    )(degp, x, W1)


def _tc_mid(part, zp, dinv, b, W, BN):
    G, F, N = part.shape
    Hn = W.shape[1]
    grid = (pl.cdiv(N, BN),)

    def body(part_ref, zp_ref, dinv_ref, b_ref, w_ref, out_ref):
        red = jnp.sum(part_ref[...], axis=0)
        dinv = dinv_ref[...]
        h = jnp.maximum(dinv * (red + zp_ref[...]) + b_ref[...], 0.0)
        z = lax.dot_general(w_ref[...], h, (((0,), (0,)), ((), ())),
                            preferred_element_type=jnp.float32)
        out_ref[...] = z * dinv

    return pl.pallas_call(
        body,
        grid=grid,
        in_specs=[
            pl.BlockSpec((G, F, BN), lambda i: (0, 0, i)),
            pl.BlockSpec((F, BN), lambda i: (0, i)),
            pl.BlockSpec((1, BN), lambda i: (0, i)),
            pl.BlockSpec((F, 1), lambda i: (0, 0)),
            pl.BlockSpec((F, Hn), lambda i: (0, 0)),
        ],
        out_specs=pl.BlockSpec((Hn, BN), lambda i: (0, i)),
        out_shape=jax.ShapeDtypeStruct((Hn, N), jnp.float32),
    )(part, zp, dinv, b, W)


def _tc_tail(part, zp, dinv, b, BN):
    G, C, N = part.shape
    grid = (pl.cdiv(N, BN),)

    def body(part_ref, zp_ref, dinv_ref, b_ref, out_ref):
        red = jnp.sum(part_ref[...], axis=0)
        t = dinv_ref[...] * (red + zp_ref[...]) + b_ref[...]
        m = jnp.max(t, axis=0, keepdims=True)
        u = t - m
        lse = jnp.log(jnp.sum(jnp.exp(u), axis=0, keepdims=True))
        out_ref[...] = u - lse

    return pl.pallas_call(
        body,
        grid=grid,
        in_specs=[
            pl.BlockSpec((G, C, BN), lambda i: (0, 0, i)),
            pl.BlockSpec((C, BN), lambda i: (0, i)),
            pl.BlockSpec((1, BN), lambda i: (0, i)),
            pl.BlockSpec((C, 1), lambda i: (0, 0)),
        ],
        out_specs=pl.BlockSpec((C, BN), lambda i: (0, i)),
        out_shape=jax.ShapeDtypeStruct((C, N), jnp.float32),
    )(part, zp, dinv, b)


def kernel(x, edge_index, edge_weight, W1, b1, W2, b2, W3, b3):
    N, D = x.shape
    E = edge_index.shape[1]
    H1 = W1.shape[1]
    H2 = W2.shape[1]
    C = W3.shape[1]
    BN = 2048

    src = edge_index[0]
    dst = edge_index[1]

    degp, pk = _make_deg(N, E, E // NW)(src, dst, edge_weight)
    z1p, dinv = _tc_head(degp, x, W1, BN)

    agg_h = _make_agg(N, E, H1, 4, 4, 4000)
    p1 = agg_h(z1p, pk, edge_weight)
    z2p = _tc_mid(p1, z1p, dinv, b1.reshape(H1, 1), W2, BN)
    p2 = agg_h(z2p, pk, edge_weight)
    z3p = _tc_mid(p2, z2p, dinv, b2.reshape(H2, 1), W3, BN)

    p3 = _make_agg(N, E, C, C, NW, E // NW)(z3p, pk, edge_weight)
    outT = _tc_tail(p3, z3p, dinv, b3.reshape(C, 1), BN)
    return outT.T


# speedup vs baseline: 1.1000x; 1.1000x over previous
import functools

import jax
import jax.numpy as jnp
from jax import lax
from jax.experimental import pallas as pl
from jax.experimental.pallas import tpu as pltpu
from jax.experimental.pallas import tpu_sc as plsc

NW = 32
LANES = 16


def _worker_id():
    c = lax.axis_index("c")
    s = lax.axis_index("s")
    return s * 2 + c


def _make_deg(N, E, CH):
    PER = E // NW
    assert CH % LANES == 0 and N % LANES == 0 and PER % CH == 0
    mesh = plsc.VectorSubcoreMesh(core_axis_name="c", subcore_axis_name="s")

    @functools.partial(
        pl.kernel,
        out_type=[
            jax.ShapeDtypeStruct((NW, N), jnp.float32),
            jax.ShapeDtypeStruct((E,), jnp.int32),
        ],
        mesh=mesh,
        compiler_params=pltpu.CompilerParams(needs_layout_passes=False),
        scratch_types=[
            pltpu.VMEM((N,), jnp.float32),
            pltpu.VMEM((CH,), jnp.int32),
            pltpu.VMEM((CH,), jnp.int32),
            pltpu.VMEM((CH,), jnp.float32),
            pltpu.VMEM((CH,), jnp.int32),
            pltpu.SemaphoreType.DMA,
        ],
    )
    def deg_kernel(src, dst, ew, degp, pk, acc, src_v, dst_v, ew_v, pk_v, sem):
        wid = _worker_id()
        ebase = wid * PER
        copies = [
            pltpu.make_async_copy(src.at[pl.ds(ebase, CH)], src_v, sem),
            pltpu.make_async_copy(dst.at[pl.ds(ebase, CH)], dst_v, sem),
            pltpu.make_async_copy(ew.at[pl.ds(ebase, CH)], ew_v, sem),
        ]
        for cp in copies:
            cp.start()

        zero = jnp.zeros((LANES,), jnp.float32)

        @plsc.parallel_loop(0, N, LANES, unroll=8)
        def _zero(i):
            acc[pl.ds(i, LANES)] = zero

        for cp in copies:
            cp.wait()

        @plsc.parallel_loop(0, CH, LANES, unroll=8)
        def _grp(i):
            sv = src_v[pl.ds(i, LANES)]
            dv = dst_v[pl.ds(i, LANES)]
            wv = ew_v[pl.ds(i, LANES)]
            pk_v[pl.ds(i, LANES)] = jnp.bitwise_or(
                jnp.left_shift(dv, 16), sv)
            plsc.addupdate_scatter(acc, [dv], wv)

        pltpu.sync_copy(pk_v, pk.at[pl.ds(ebase, CH)])
        pltpu.sync_copy(acc, degp.at[wid])

    return deg_kernel


def _make_agg(N, E, F, FB, G, CH):
    NFB = F // FB
    assert G * NFB == NW
    assert CH % LANES == 0 and N % LANES == 0
    NP = FB // 2
    HALF = F // 2
    PER = E // G
    NCH = PER // CH
    NB = 2 if NCH > 1 else 1
    assert NCH % NB == 0
    mesh = plsc.VectorSubcoreMesh(core_axis_name="c", subcore_axis_name="s")
    scratch = (
        [pltpu.VMEM((N,), jnp.int32) for _ in range(NP)]
        + [pltpu.VMEM((N,), jnp.float32) for _ in range(FB)]
        + [pltpu.VMEM((CH,), jnp.int32) for _ in range(NB)]
        + [pltpu.VMEM((CH,), jnp.float32) for _ in range(NB)]
        + [pltpu.SemaphoreType.DMA for _ in range(NB + 1)]
    )

    @functools.partial(
        pl.kernel,
        out_type=jax.ShapeDtypeStruct((G, F, N), jnp.float32),
        mesh=mesh,
        compiler_params=pltpu.CompilerParams(needs_layout_passes=False),
        scratch_types=scratch,
    )
    def agg_kernel(hTk, pk, ew, part, *scr):
        tbls = scr[:NP]
        accs = scr[NP:NP + FB]
        pk_v = scr[NP + FB:NP + FB + NB]
        ew_v = scr[NP + FB + NB:NP + FB + 2 * NB]
        sems = scr[NP + FB + 2 * NB:NP + FB + 3 * NB]
        tsem = scr[NP + FB + 3 * NB]
        wid = _worker_id()
        g = wid % G
        fb = wid // G
        pbase = fb * NP
        ebase = g * PER

        def edge_copies(ci, b):
            off = ebase + ci * CH
            return [
                pltpu.make_async_copy(pk.at[pl.ds(off, CH)], pk_v[b], sems[b]),
                pltpu.make_async_copy(ew.at[pl.ds(off, CH)], ew_v[b], sems[b]),
            ]

        def issue(ci, b):
            for cp in edge_copies(ci, b):
                cp.start()

        def wait(ci, b):
            for cp in edge_copies(ci, b):
                cp.wait()

        for b in range(NB):
            issue(b, b)
        tcopies = [
            pltpu.make_async_copy(hTk.at[pbase + j], tbls[j], tsem)
            for j in range(NP)
        ]
        for cp in tcopies:
            cp.start()

        zero = jnp.zeros((LANES,), jnp.float32)

        @plsc.parallel_loop(0, N, LANES, unroll=4)
        def _zero(i):
            for k in range(FB):
                accs[k][pl.ds(i, LANES)] = zero

        for cp in tcopies:
            cp.wait()

        def process(b):
            @plsc.parallel_loop(0, CH, LANES, unroll=8)
            def _grp(i):
                pkv = pk_v[b][pl.ds(i, LANES)]
                sv = jnp.bitwise_and(pkv, jnp.int32(0xFFFF))
                dv = jnp.right_shift(pkv, 16)
                wv = ew_v[b][pl.ds(i, LANES)]
                for j in range(NP):
                    gv = plsc.load_gather(tbls[j], [sv])
                    lo = plsc.bitcast(jnp.left_shift(gv, 16), jnp.float32)
                    hi = plsc.bitcast(
                        jnp.bitwise_and(gv, jnp.int32(-65536)), jnp.float32)
                    plsc.addupdate_scatter(accs[2 * j], [dv], lo * wv)
                    plsc.addupdate_scatter(accs[2 * j + 1], [dv], hi * wv)

        if NB == 1:
            def chunk(ci, _):
                wait(ci, 0)
                process(0)

                @pl.when(ci + 1 < NCH)
                def _():
                    issue(ci + 1, 0)
                return 0

            lax.fori_loop(0, NCH, chunk, 0)
        else:
            def pair(pi, _):
                ci0 = pi * 2
                for b in range(2):
                    wait(ci0 + b, b)
                    process(b)

                    @pl.when(ci0 + b + 2 < NCH)
                    def _():
                        issue(ci0 + b + 2, b)
                return 0

            lax.fori_loop(0, NCH // 2, pair, 0)

        for j in range(NP):
            pltpu.sync_copy(accs[2 * j], part.at[g, pbase + j])
            pltpu.sync_copy(accs[2 * j + 1], part.at[g, pbase + j + HALF])

    return agg_kernel


def _pack_pairs(zp):
    half = zp.shape[0] // 2
    lo = lax.bitcast_convert_type(
        zp[:half].astype(jnp.bfloat16), jnp.uint16).astype(jnp.uint32)
    hi = lax.bitcast_convert_type(
        zp[half:].astype(jnp.bfloat16), jnp.uint16).astype(jnp.uint32)
    return lax.bitcast_convert_type(
        jnp.bitwise_or(jnp.left_shift(hi, 16), lo), jnp.int32)


def _tc_head(degp, x, W1, BN):
    NWp, N = degp.shape
    D = x.shape[1]
    H = W1.shape[1]
    grid = (pl.cdiv(N, BN),)

    def body(degp_ref, x_ref, w_ref, zp_ref, dinv_ref, zpk_ref):
        deg = jnp.sum(degp_ref[...], axis=0, keepdims=True) + 1.0
        dinv = lax.rsqrt(deg)
        z = lax.dot_general(w_ref[...], x_ref[...], (((0,), (1,)), ((), ())),
                            preferred_element_type=jnp.float32)
        zp = z * dinv
        zp_ref[...] = zp
        dinv_ref[...] = dinv
        zpk_ref[...] = _pack_pairs(zp)

    return pl.pallas_call(
        body,
        grid=grid,
        in_specs=[
            pl.BlockSpec((NWp, BN), lambda i: (0, i)),
            pl.BlockSpec((BN, D), lambda i: (i, 0)),
            pl.BlockSpec((D, H), lambda i: (0, 0)),
        ],
        out_specs=[
            pl.BlockSpec((H, BN), lambda i: (0, i)),
            pl.BlockSpec((1, BN), lambda i: (0, i)),
            pl.BlockSpec((H // 2, BN), lambda i: (0, i)),
        ],
        out_shape=[
            jax.ShapeDtypeStruct((H, N), jnp.float32),
            jax.ShapeDtypeStruct((1, N), jnp.float32),
            jax.ShapeDtypeStruct((H // 2, N), jnp.int32),
        ],
    )(degp, x, W1)


def _tc_mid(part, zp, dinv, b, W, BN):
    G, F, N = part.shape
    Hn = W.shape[1]
    grid = (pl.cdiv(N, BN),)

    def body(part_ref, zp_ref, dinv_ref, b_ref, w_ref, out_ref, outk_ref):
        red = jnp.sum(part_ref[...], axis=0)
        dinv = dinv_ref[...]
        h = jnp.maximum(dinv * (red + zp_ref[...]) + b_ref[...], 0.0)
        z = lax.dot_general(w_ref[...], h, (((0,), (0,)), ((), ())),
                            preferred_element_type=jnp.float32)
        zp_next = z * dinv
        out_ref[...] = zp_next
        outk_ref[...] = _pack_pairs(zp_next)

    return pl.pallas_call(
        body,
        grid=grid,
        in_specs=[
            pl.BlockSpec((G, F, BN), lambda i: (0, 0, i)),
            pl.BlockSpec((F, BN), lambda i: (0, i)),
            pl.BlockSpec((1, BN), lambda i: (0, i)),
            pl.BlockSpec((F, 1), lambda i: (0, 0)),
            pl.BlockSpec((F, Hn), lambda i: (0, 0)),
        ],
        out_specs=[
            pl.BlockSpec((Hn, BN), lambda i: (0, i)),
            pl.BlockSpec((Hn // 2, BN), lambda i: (0, i)),
        ],
        out_shape=[
            jax.ShapeDtypeStruct((Hn, N), jnp.float32),
            jax.ShapeDtypeStruct((Hn // 2, N), jnp.int32),
        ],
    )(part, zp, dinv, b, W)


def _tc_tail(part, zp, dinv, b, BN):
    G, C, N = part.shape
    grid = (pl.cdiv(N, BN),)

    def body(part_ref, zp_ref, dinv_ref, b_ref, out_ref):
        red = jnp.sum(part_ref[...], axis=0)
        t = dinv_ref[...] * (red + zp_ref[...]) + b_ref[...]
        m = jnp.max(t, axis=0, keepdims=True)
        u = t - m
        lse = jnp.log(jnp.sum(jnp.exp(u), axis=0, keepdims=True))
        out_ref[...] = u - lse

    return pl.pallas_call(
        body,
        grid=grid,
        in_specs=[
            pl.BlockSpec((G, C, BN), lambda i: (0, 0, i)),
            pl.BlockSpec((C, BN), lambda i: (0, i)),
            pl.BlockSpec((1, BN), lambda i: (0, i)),
            pl.BlockSpec((C, 1), lambda i: (0, 0)),
        ],
        out_specs=pl.BlockSpec((C, BN), lambda i: (0, i)),
        out_shape=jax.ShapeDtypeStruct((C, N), jnp.float32),
    )(part, zp, dinv, b)


def kernel(x, edge_index, edge_weight, W1, b1, W2, b2, W3, b3):
    N, D = x.shape
    E = edge_index.shape[1]
    H1 = W1.shape[1]
    H2 = W2.shape[1]
    C = W3.shape[1]
    BN = 2048

    src = edge_index[0]
    dst = edge_index[1]

    degp, pk = _make_deg(N, E, E // NW)(src, dst, edge_weight)
    z1p, dinv, z1k = _tc_head(degp, x, W1, BN)

    agg_h = _make_agg(N, E, H1, 4, 4, 4000)
    p1 = agg_h(z1k, pk, edge_weight)
    z2p, z2k = _tc_mid(p1, z1p, dinv, b1.reshape(H1, 1), W2, BN)
    p2 = agg_h(z2k, pk, edge_weight)
    z3p, z3k = _tc_mid(p2, z2p, dinv, b2.reshape(H2, 1), W3, BN)

    p3 = _make_agg(N, E, C, C, NW, E // NW)(z3k, pk, edge_weight)
    outT = _tc_tail(p3, z3p, dinv, b3.reshape(C, 1), BN)
    return outT.T
